# baseline (speedup 1.0000x reference)
import functools

import jax
import jax.numpy as jnp
from jax import lax
from jax.experimental import pallas as pl
from jax.experimental.pallas import tpu as pltpu
from jax.experimental.pallas import tpu_sc as plsc

N = 10000
E = 160000
F = 256
DE = 16
FH = F // 2
NC = 2
NS = 16
LANES = 16

EC_AGG = 64
EDGES_PER_TILE = E // NS
NBUF = 3
N_FULL = EDGES_PER_TILE // EC_AGG
N_WAVES = N_FULL // NBUF
TAIL_E = EDGES_PER_TILE - N_FULL * EC_AGG
ROWS_PER_TILE = 624
TAIL_BASE = NS * ROWS_PER_TILE
TAIL_ROWS = N - TAIL_BASE


def _agg_body(h_lo, h_hi, src, dst, an, agg_lo, agg_hi,
              src_v, dst_v, r0, r1, r2, a0, a1, a2, shared,
              g0, g1, g2, q0, q1, q2, ssem):
    c = lax.axis_index("c")
    s = lax.axis_index("s")
    slots = (r0, r1, r2)
    anbs = (a0, a1, a2)
    gsems = (g0, g1, g2)
    asems = (q0, q1, q2)

    ebase = s * EDGES_PER_TILE
    lsrc = pltpu.async_copy(src.at[pl.ds(ebase, EDGES_PER_TILE)], src_v, g0)
    ldst = pltpu.async_copy(dst.at[pl.ds(ebase, EDGES_PER_TILE)], dst_v, g1)

    @pl.loop(0, EC_AGG)
    def _zr(r):
        for j in range(FH // LANES):
            r2[r, pl.ds(j * LANES, LANES)] = jnp.zeros((LANES,), jnp.float32)

    for t in range(ROWS_PER_TILE // EC_AGG):
        pltpu.sync_copy(
            r2, shared.at[pl.ds(s * ROWS_PER_TILE + t * EC_AGG, EC_AGG)])
    rem = ROWS_PER_TILE - (ROWS_PER_TILE // EC_AGG) * EC_AGG
    pltpu.sync_copy(
        r2.at[pl.ds(0, rem)],
        shared.at[pl.ds(s * ROWS_PER_TILE + ROWS_PER_TILE - rem, rem)])

    @pl.when(s == NS - 1)
    def _():
        pltpu.sync_copy(r2.at[pl.ds(0, TAIL_ROWS)],
                        shared.at[pl.ds(TAIL_BASE, TAIL_ROWS)])

    lsrc.wait()
    ldst.wait()
    plsc.subcore_barrier()

    def scale_rows(slot, anb, nrows):
        @pl.loop(0, nrows)
        def _row(r):
            a = plsc.load_gather(anb, [jnp.full((LANES,), r, jnp.int32)])
            for j in range(FH // LANES):
                sl = pl.ds(j * LANES, LANES)
                slot[r, sl] = slot[r, sl] * a

    def chunk_loop(tab):
        @pl.loop(0, N_WAVES)
        def _wave(u):
            gh = []
            for b in range(NBUF):
                i = u * NBUF + b
                gh.append((
                    pltpu.async_copy(
                        tab.at[src_v.at[pl.ds(i * EC_AGG, EC_AGG)]],
                        slots[b], gsems[b]),
                    pltpu.async_copy(
                        an.at[pl.ds(ebase + i * EC_AGG, EC_AGG)],
                        anbs[b], asems[b])))
            sh = []
            for b in range(NBUF):
                i = u * NBUF + b
                cb = i * EC_AGG
                gh[b][0].wait()
                gh[b][1].wait()
                scale_rows(slots[b], anbs[b], EC_AGG)
                for q in range(EC_AGG // LANES):
                    dvec = dst_v[pl.ds(cb + q * LANES, LANES)]
                    sh.append(pltpu.async_copy(
                        slots[b].at[pl.ds(q * LANES, LANES)],
                        shared.at[dvec], ssem, add=True))
            for h in sh:
                h.wait()

        tb = N_FULL * EC_AGG
        pltpu.async_copy(
            tab.at[src_v.at[pl.ds(tb, TAIL_E)]],
            r0.at[pl.ds(0, TAIL_E)], g0).wait()
        pltpu.async_copy(an.at[pl.ds(ebase + tb, TAIL_E)], a0.at[pl.ds(0, TAIL_E)],
                         q0).wait()
        scale_rows(r0, a0, TAIL_E)
        dvec = dst_v[pl.ds(tb, LANES)]
        pltpu.sync_copy(r0.at[pl.ds(0, LANES)], shared.at[dvec], add=True)

    @pl.when(c == 0)
    def _():
        chunk_loop(h_lo)

    @pl.when(c == 1)
    def _():
        chunk_loop(h_hi)

    plsc.subcore_barrier()

    def writeout(out):
        sl = pl.ds(s * ROWS_PER_TILE, ROWS_PER_TILE)
        pltpu.sync_copy(shared.at[sl], out.at[sl])

        @pl.when(s == NS - 1)
        def _():
            tl = pl.ds(TAIL_BASE, TAIL_ROWS)
            pltpu.sync_copy(shared.at[tl], out.at[tl])

    @pl.when(c == 0)
    def _():
        writeout(agg_lo)

    @pl.when(c == 1)
    def _():
        writeout(agg_hi)


_agg_call = pl.kernel(
    _agg_body,
    out_type=(jax.ShapeDtypeStruct((N, FH), jnp.float32),
              jax.ShapeDtypeStruct((N, FH), jnp.float32)),
    mesh=plsc.VectorSubcoreMesh(core_axis_name="c", subcore_axis_name="s"),
    compiler_params=pltpu.CompilerParams(needs_layout_passes=False),
    scratch_types=[
        pltpu.VMEM((EDGES_PER_TILE,), jnp.int32),
        pltpu.VMEM((EDGES_PER_TILE,), jnp.int32),
        pltpu.VMEM((EC_AGG, FH), jnp.float32),
        pltpu.VMEM((EC_AGG, FH), jnp.float32),
        pltpu.VMEM((EC_AGG, FH), jnp.float32),
        pltpu.VMEM((EC_AGG,), jnp.float32),
        pltpu.VMEM((EC_AGG,), jnp.float32),
        pltpu.VMEM((EC_AGG,), jnp.float32),
        pltpu.VMEM_SHARED((N, FH), jnp.float32),
        pltpu.SemaphoreType.DMA,
        pltpu.SemaphoreType.DMA,
        pltpu.SemaphoreType.DMA,
        pltpu.SemaphoreType.DMA,
        pltpu.SemaphoreType.DMA,
        pltpu.SemaphoreType.DMA,
        pltpu.SemaphoreType.DMA,
    ],
)

DH = DE // 2
E2 = E // 2
EDGES_PER_SUB = E // NS
G_BLK = 2000
G_NBLK = EDGES_PER_SUB // G_BLK
G_GROUPS = G_BLK // LANES


def _egather_body(hxl, hxh, src, dst, out0, out1, src_v, dst_v, tab_v, ob_v):
    c = lax.axis_index("c")
    s = lax.axis_index("s")
    iota = lax.iota(jnp.int32, 16)
    base16 = lax.shift_right_logical(iota, 3)
    col = lax.bitwise_and(iota, 7)

    pltpu.sync_copy(src.at[pl.ds(s * EDGES_PER_SUB, EDGES_PER_SUB)], src_v)
    pltpu.sync_copy(dst.at[pl.ds(s * EDGES_PER_SUB, EDGES_PER_SUB)], dst_v)

    def run(tab, out):
        pltpu.sync_copy(tab, tab_v)

        @pl.loop(0, G_NBLK)
        def _blk(b):
            @pl.loop(0, G_GROUPS)
            def _grp(g):
                eb = (b * G_GROUPS + g) * LANES
                for j in range(LANES // 2):
                    idx = base16 + (eb + 2 * j)
                    rows_s = plsc.load_gather(src_v, [idx])
                    rows_d = plsc.load_gather(dst_v, [idx])
                    fs = lax.bitwise_or(lax.shift_left(rows_s, 3), col)
                    fd = lax.bitwise_or(lax.shift_left(rows_d, 3), col)
                    vs = plsc.load_gather(tab_v, [fs])
                    vd = plsc.load_gather(tab_v, [fd])
                    ob_v[pl.ds((g * (LANES // 2) + j) * LANES, LANES)] = vs + vd

            pltpu.sync_copy(
                ob_v, out.at[pl.ds((s * EDGES_PER_SUB + b * G_BLK) * DH,
                                   G_BLK * DH)])

    @pl.when(c == 0)
    def _():
        run(hxl, out0)

    @pl.when(c == 1)
    def _():
        run(hxh, out1)


_egather_call = pl.kernel(
    _egather_body,
    out_type=(jax.ShapeDtypeStruct((E * DH,), jnp.float32),
              jax.ShapeDtypeStruct((E * DH,), jnp.float32)),
    mesh=plsc.VectorSubcoreMesh(core_axis_name="c", subcore_axis_name="s"),
    compiler_params=pltpu.CompilerParams(needs_layout_passes=False),
    scratch_types=[
        pltpu.VMEM((EDGES_PER_SUB,), jnp.int32),
        pltpu.VMEM((EDGES_PER_SUB,), jnp.int32),
        pltpu.VMEM((N * DH,), jnp.float32),
        pltpu.VMEM((G_BLK * DH,), jnp.float32),
    ],
)

RB = 1000


def _dense_body(last, agg_lo_ref, agg_hi_ref, h_lo_ref, h_hi_ref,
                w_ref, wg_ref, ug_ref, wxe_ref,
                hn_lo_ref, hn_hi_ref, hxl_ref, hxh_ref):
    h = jnp.concatenate([h_lo_ref[...], h_hi_ref[...]], axis=1)
    agg = jnp.concatenate([agg_lo_ref[...], agg_hi_ref[...]], axis=1)
    hl = jnp.dot(agg, w_ref[...], preferred_element_type=jnp.float32)
    g = jax.nn.sigmoid(
        jnp.dot(h, wg_ref[...], preferred_element_type=jnp.float32)
        + jnp.dot(hl, ug_ref[...], preferred_element_type=jnp.float32))
    hn = g * hl + (1.0 - g) * h
    mu = jnp.mean(hn, axis=-1, keepdims=True)
    var = jnp.mean((hn - mu) * (hn - mu), axis=-1, keepdims=True)
    hn = (hn - mu) * lax.rsqrt(var + 1e-5)
    hx = jnp.dot(hn, wxe_ref[...], preferred_element_type=jnp.float32)
    hxl_ref[...] = hx[:, :DH]
    hxh_ref[...] = hx[:, DH:]
    if last:
        hn = jnp.maximum(hn, 0.0)
    hn_lo_ref[...] = hn[:, :FH]
    hn_hi_ref[...] = hn[:, FH:]


def _make_dense(last):
    return pl.pallas_call(
        functools.partial(_dense_body, last),
        grid=(N // RB,),
        in_specs=[
            pl.BlockSpec((RB, FH), lambda i: (i, 0)),
            pl.BlockSpec((RB, FH), lambda i: (i, 0)),
            pl.BlockSpec((RB, FH), lambda i: (i, 0)),
            pl.BlockSpec((RB, FH), lambda i: (i, 0)),
            pl.BlockSpec((F, F), lambda i: (0, 0)),
            pl.BlockSpec((F, F), lambda i: (0, 0)),
            pl.BlockSpec((F, F), lambda i: (0, 0)),
            pl.BlockSpec((F, DE), lambda i: (0, 0)),
        ],
        out_specs=(
            pl.BlockSpec((RB, FH), lambda i: (i, 0)),
            pl.BlockSpec((RB, FH), lambda i: (i, 0)),
            pl.BlockSpec((RB, DH), lambda i: (i, 0)),
            pl.BlockSpec((RB, DH), lambda i: (i, 0)),
        ),
        out_shape=(
            jax.ShapeDtypeStruct((N, FH), jnp.float32),
            jax.ShapeDtypeStruct((N, FH), jnp.float32),
            jax.ShapeDtypeStruct((N, DH), jnp.float32),
            jax.ShapeDtypeStruct((N, DH), jnp.float32),
        ),
    )


_dense_calls = (_make_dense(False), _make_dense(True))

EB2 = 8000


def _edge_body(e_ref, c0_ref, c1_ref, we_ref, out_ref):
    c0 = c0_ref[...]
    c1 = c1_ref[...]
    contrib = jnp.concatenate(
        [c0[:, :DH], c1[:, :DH], c0[:, DH:], c1[:, DH:]], axis=1)
    out_ref[...] = jnp.maximum(
        jnp.dot(e_ref[...], we_ref[...], preferred_element_type=jnp.float32)
        + contrib, 0.0)


_edge_call = pl.pallas_call(
    _edge_body,
    grid=(E2 // EB2,),
    in_specs=[
        pl.BlockSpec((EB2, 2 * DE), lambda i: (i, 0)),
        pl.BlockSpec((EB2, DE), lambda i: (i, 0)),
        pl.BlockSpec((EB2, DE), lambda i: (i, 0)),
        pl.BlockSpec((2 * DE, 2 * DE), lambda i: (0, 0)),
    ],
    out_specs=pl.BlockSpec((EB2, 2 * DE), lambda i: (i, 0)),
    out_shape=jax.ShapeDtypeStruct((E2, 2 * DE), jnp.float32),
)


@jax.jit
def _run(x, src, dst, an, e0, W, Wg, Ug, We, Wxe):
    n_layers = W.shape[0]
    h_lo = x[:, :FH]
    h_hi = x[:, FH:]
    ep = e0.reshape(E2, 2 * DE)
    eye2 = jnp.eye(2, dtype=jnp.float32)
    for l in range(n_layers):
        last = l == n_layers - 1
        agg_lo, agg_hi = _agg_call(h_lo, h_hi, src, dst, an)
        h_lo, h_hi, hxl, hxh = _dense_calls[int(last)](
            agg_lo, agg_hi, h_lo, h_hi, W[l], Wg[l], Ug[l], Wxe[l])
        c0, c1 = _egather_call(hxl.reshape(-1), hxh.reshape(-1), src, dst)
        an, c0 = lax.optimization_barrier((an, c0))
        ep = _edge_call(ep, c0.reshape(E2, DE), c1.reshape(E2, DE),
                        jnp.kron(eye2, We[l]))
    h = jnp.concatenate([h_lo, h_hi], axis=1)
    return h, ep.reshape(E, DE)


def kernel(x, edge_index, A_norm, edge_attr, W, Wg, Ug, We, Wxe):
    src = edge_index[0].astype(jnp.int32)
    dst = edge_index[1].astype(jnp.int32)
    return _run(x, src, dst, A_norm, edge_attr, W, Wg, Ug, We, Wxe)


# speedup vs baseline: 1.1900x; 1.1900x over previous
import functools

import jax
import jax.numpy as jnp
from jax import lax
from jax.experimental import pallas as pl
from jax.experimental.pallas import tpu as pltpu
from jax.experimental.pallas import tpu_sc as plsc

N = 10000
E = 160000
F = 256
DE = 16
FH = F // 2
NC = 2
NS = 16
LANES = 16

EC_AGG = 64
EDGES_PER_TILE = E // NS
NBUF = 3
N_FULL = EDGES_PER_TILE // EC_AGG
N_WAVES = N_FULL // NBUF
TAIL_E = EDGES_PER_TILE - N_FULL * EC_AGG
ROWS_PER_TILE = 624
TAIL_BASE = NS * ROWS_PER_TILE
TAIL_ROWS = N - TAIL_BASE


def _agg_body(h_lo, h_hi, src, dst, an, agg_lo, agg_hi,
              src_v, dst_v, r0, r1, r2, a0, a1, a2, shared,
              g0, g1, g2, q0, q1, q2, ssem):
    c = lax.axis_index("c")
    s = lax.axis_index("s")
    slots = (r0, r1, r2)
    anbs = (a0, a1, a2)
    gsems = (g0, g1, g2)
    asems = (q0, q1, q2)

    ebase = s * EDGES_PER_TILE
    lsrc = pltpu.async_copy(src.at[pl.ds(ebase, EDGES_PER_TILE)], src_v, g0)
    ldst = pltpu.async_copy(dst.at[pl.ds(ebase, EDGES_PER_TILE)], dst_v, g1)

    @pl.loop(0, EC_AGG)
    def _zr(r):
        for j in range(FH // LANES):
            r2[r, pl.ds(j * LANES, LANES)] = jnp.zeros((LANES,), jnp.float32)

    for t in range(ROWS_PER_TILE // EC_AGG):
        pltpu.sync_copy(
            r2, shared.at[pl.ds(s * ROWS_PER_TILE + t * EC_AGG, EC_AGG)])
    rem = ROWS_PER_TILE - (ROWS_PER_TILE // EC_AGG) * EC_AGG
    pltpu.sync_copy(
        r2.at[pl.ds(0, rem)],
        shared.at[pl.ds(s * ROWS_PER_TILE + ROWS_PER_TILE - rem, rem)])

    @pl.when(s == NS - 1)
    def _():
        pltpu.sync_copy(r2.at[pl.ds(0, TAIL_ROWS)],
                        shared.at[pl.ds(TAIL_BASE, TAIL_ROWS)])

    lsrc.wait()
    ldst.wait()
    plsc.subcore_barrier()

    def scale_rows(slot, anb, nrows):
        @pl.loop(0, nrows)
        def _row(r):
            a = plsc.load_gather(anb, [jnp.full((LANES,), r, jnp.int32)])
            for j in range(FH // LANES):
                sl = pl.ds(j * LANES, LANES)
                slot[r, sl] = slot[r, sl] * a

    def chunk_loop(tab):
        for b in range(NBUF):
            pltpu.async_copy(
                tab.at[src_v.at[pl.ds(b * EC_AGG, EC_AGG)]],
                slots[b], gsems[b])
            pltpu.async_copy(
                an.at[pl.ds(ebase + b * EC_AGG, EC_AGG)], anbs[b], asems[b])

        @pl.loop(0, N_WAVES)
        def _wave(u):
            for b in range(NBUF):
                i = u * NBUF + b
                cb = i * EC_AGG
                pltpu.make_async_copy(
                    tab.at[src_v.at[pl.ds(0, EC_AGG)]],
                    slots[b], gsems[b]).wait()
                pltpu.make_async_copy(
                    an.at[pl.ds(ebase, EC_AGG)], anbs[b], asems[b]).wait()
                scale_rows(slots[b], anbs[b], EC_AGG)
                sh = []
                for q in range(EC_AGG // LANES):
                    dvec = dst_v[pl.ds(cb + q * LANES, LANES)]
                    sh.append(pltpu.async_copy(
                        slots[b].at[pl.ds(q * LANES, LANES)],
                        shared.at[dvec], ssem, add=True))
                for h in sh:
                    h.wait()

                @pl.when(i + NBUF < N_FULL)
                def _(b=b, i=i):
                    pltpu.async_copy(
                        tab.at[src_v.at[pl.ds((i + NBUF) * EC_AGG, EC_AGG)]],
                        slots[b], gsems[b])
                    pltpu.async_copy(
                        an.at[pl.ds(ebase + (i + NBUF) * EC_AGG, EC_AGG)],
                        anbs[b], asems[b])

        tb = N_FULL * EC_AGG
        pltpu.async_copy(
            tab.at[src_v.at[pl.ds(tb, TAIL_E)]],
            r0.at[pl.ds(0, TAIL_E)], g0).wait()
        pltpu.async_copy(an.at[pl.ds(ebase + tb, TAIL_E)], a0.at[pl.ds(0, TAIL_E)],
                         q0).wait()
        scale_rows(r0, a0, TAIL_E)
        dvec = dst_v[pl.ds(tb, LANES)]
        pltpu.sync_copy(r0.at[pl.ds(0, LANES)], shared.at[dvec], add=True)

    @pl.when(c == 0)
    def _():
        chunk_loop(h_lo)

    @pl.when(c == 1)
    def _():
        chunk_loop(h_hi)

    plsc.subcore_barrier()

    def writeout(out):
        sl = pl.ds(s * ROWS_PER_TILE, ROWS_PER_TILE)
        pltpu.sync_copy(shared.at[sl], out.at[sl])

        @pl.when(s == NS - 1)
        def _():
            tl = pl.ds(TAIL_BASE, TAIL_ROWS)
            pltpu.sync_copy(shared.at[tl], out.at[tl])

    @pl.when(c == 0)
    def _():
        writeout(agg_lo)

    @pl.when(c == 1)
    def _():
        writeout(agg_hi)


_agg_call = pl.kernel(
    _agg_body,
    out_type=(jax.ShapeDtypeStruct((N, FH), jnp.float32),
              jax.ShapeDtypeStruct((N, FH), jnp.float32)),
    mesh=plsc.VectorSubcoreMesh(core_axis_name="c", subcore_axis_name="s"),
    compiler_params=pltpu.CompilerParams(needs_layout_passes=False),
    scratch_types=[
        pltpu.VMEM((EDGES_PER_TILE,), jnp.int32),
        pltpu.VMEM((EDGES_PER_TILE,), jnp.int32),
        pltpu.VMEM((EC_AGG, FH), jnp.float32),
        pltpu.VMEM((EC_AGG, FH), jnp.float32),
        pltpu.VMEM((EC_AGG, FH), jnp.float32),
        pltpu.VMEM((EC_AGG,), jnp.float32),
        pltpu.VMEM((EC_AGG,), jnp.float32),
        pltpu.VMEM((EC_AGG,), jnp.float32),
        pltpu.VMEM_SHARED((N, FH), jnp.float32),
        pltpu.SemaphoreType.DMA,
        pltpu.SemaphoreType.DMA,
        pltpu.SemaphoreType.DMA,
        pltpu.SemaphoreType.DMA,
        pltpu.SemaphoreType.DMA,
        pltpu.SemaphoreType.DMA,
        pltpu.SemaphoreType.DMA,
    ],
)

DH = DE // 2
E2 = E // 2
EDGES_PER_SUB = E // NS
G_BLK = 2000
G_NBLK = EDGES_PER_SUB // G_BLK
G_GROUPS = G_BLK // LANES


def _egather_body(hxl, hxh, src, dst, out0, out1, src_v, dst_v, tab_v, ob_v):
    c = lax.axis_index("c")
    s = lax.axis_index("s")
    iota = lax.iota(jnp.int32, 16)
    base16 = lax.shift_right_logical(iota, 3)
    col = lax.bitwise_and(iota, 7)

    pltpu.sync_copy(src.at[pl.ds(s * EDGES_PER_SUB, EDGES_PER_SUB)], src_v)
    pltpu.sync_copy(dst.at[pl.ds(s * EDGES_PER_SUB, EDGES_PER_SUB)], dst_v)

    def run(tab, out):
        pltpu.sync_copy(tab, tab_v)

        @pl.loop(0, G_NBLK)
        def _blk(b):
            @pl.loop(0, G_GROUPS)
            def _grp(g):
                eb = (b * G_GROUPS + g) * LANES
                for j in range(LANES // 2):
                    idx = base16 + (eb + 2 * j)
                    rows_s = plsc.load_gather(src_v, [idx])
                    rows_d = plsc.load_gather(dst_v, [idx])
                    fs = lax.bitwise_or(lax.shift_left(rows_s, 3), col)
                    fd = lax.bitwise_or(lax.shift_left(rows_d, 3), col)
                    vs = plsc.load_gather(tab_v, [fs])
                    vd = plsc.load_gather(tab_v, [fd])
                    ob_v[pl.ds((g * (LANES // 2) + j) * LANES, LANES)] = vs + vd

            pltpu.sync_copy(
                ob_v, out.at[pl.ds((s * EDGES_PER_SUB + b * G_BLK) * DH,
                                   G_BLK * DH)])

    @pl.when(c == 0)
    def _():
        run(hxl, out0)

    @pl.when(c == 1)
    def _():
        run(hxh, out1)


_egather_call = pl.kernel(
    _egather_body,
    out_type=(jax.ShapeDtypeStruct((E * DH,), jnp.float32),
              jax.ShapeDtypeStruct((E * DH,), jnp.float32)),
    mesh=plsc.VectorSubcoreMesh(core_axis_name="c", subcore_axis_name="s"),
    compiler_params=pltpu.CompilerParams(needs_layout_passes=False),
    scratch_types=[
        pltpu.VMEM((EDGES_PER_SUB,), jnp.int32),
        pltpu.VMEM((EDGES_PER_SUB,), jnp.int32),
        pltpu.VMEM((N * DH,), jnp.float32),
        pltpu.VMEM((G_BLK * DH,), jnp.float32),
    ],
)

RB = 1000


def _dense_body(last, agg_lo_ref, agg_hi_ref, h_lo_ref, h_hi_ref,
                w_ref, wg_ref, ug_ref, wxe_ref,
                hn_lo_ref, hn_hi_ref, hxl_ref, hxh_ref):
    h = jnp.concatenate([h_lo_ref[...], h_hi_ref[...]], axis=1)
    agg = jnp.concatenate([agg_lo_ref[...], agg_hi_ref[...]], axis=1)
    hl = jnp.dot(agg, w_ref[...], preferred_element_type=jnp.float32)
    g = jax.nn.sigmoid(
        jnp.dot(h, wg_ref[...], preferred_element_type=jnp.float32)
        + jnp.dot(hl, ug_ref[...], preferred_element_type=jnp.float32))
    hn = g * hl + (1.0 - g) * h
    mu = jnp.mean(hn, axis=-1, keepdims=True)
    var = jnp.mean((hn - mu) * (hn - mu), axis=-1, keepdims=True)
    hn = (hn - mu) * lax.rsqrt(var + 1e-5)
    hx = jnp.dot(hn, wxe_ref[...], preferred_element_type=jnp.float32)
    hxl_ref[...] = hx[:, :DH]
    hxh_ref[...] = hx[:, DH:]
    if last:
        hn = jnp.maximum(hn, 0.0)
    hn_lo_ref[...] = hn[:, :FH]
    hn_hi_ref[...] = hn[:, FH:]


def _make_dense(last):
    return pl.pallas_call(
        functools.partial(_dense_body, last),
        grid=(N // RB,),
        in_specs=[
            pl.BlockSpec((RB, FH), lambda i: (i, 0)),
            pl.BlockSpec((RB, FH), lambda i: (i, 0)),
            pl.BlockSpec((RB, FH), lambda i: (i, 0)),
            pl.BlockSpec((RB, FH), lambda i: (i, 0)),
            pl.BlockSpec((F, F), lambda i: (0, 0)),
            pl.BlockSpec((F, F), lambda i: (0, 0)),
            pl.BlockSpec((F, F), lambda i: (0, 0)),
            pl.BlockSpec((F, DE), lambda i: (0, 0)),
        ],
        out_specs=(
            pl.BlockSpec((RB, FH), lambda i: (i, 0)),
            pl.BlockSpec((RB, FH), lambda i: (i, 0)),
            pl.BlockSpec((RB, DH), lambda i: (i, 0)),
            pl.BlockSpec((RB, DH), lambda i: (i, 0)),
        ),
        out_shape=(
            jax.ShapeDtypeStruct((N, FH), jnp.float32),
            jax.ShapeDtypeStruct((N, FH), jnp.float32),
            jax.ShapeDtypeStruct((N, DH), jnp.float32),
            jax.ShapeDtypeStruct((N, DH), jnp.float32),
        ),
    )


_dense_calls = (_make_dense(False), _make_dense(True))

EB2 = 8000


def _edge_body(e_ref, c0_ref, c1_ref, we_ref, out_ref):
    c0 = c0_ref[...]
    c1 = c1_ref[...]
    contrib = jnp.concatenate(
        [c0[:, :DH], c1[:, :DH], c0[:, DH:], c1[:, DH:]], axis=1)
    out_ref[...] = jnp.maximum(
        jnp.dot(e_ref[...], we_ref[...], preferred_element_type=jnp.float32)
        + contrib, 0.0)


_edge_call = pl.pallas_call(
    _edge_body,
    grid=(E2 // EB2,),
    in_specs=[
        pl.BlockSpec((EB2, 2 * DE), lambda i: (i, 0)),
        pl.BlockSpec((EB2, DE), lambda i: (i, 0)),
        pl.BlockSpec((EB2, DE), lambda i: (i, 0)),
        pl.BlockSpec((2 * DE, 2 * DE), lambda i: (0, 0)),
    ],
    out_specs=pl.BlockSpec((EB2, 2 * DE), lambda i: (i, 0)),
    out_shape=jax.ShapeDtypeStruct((E2, 2 * DE), jnp.float32),
)


@jax.jit
def _run(x, src, dst, an, e0, W, Wg, Ug, We, Wxe):
    n_layers = W.shape[0]
    h_lo = x[:, :FH]
    h_hi = x[:, FH:]
    ep = e0.reshape(E2, 2 * DE)
    eye2 = jnp.eye(2, dtype=jnp.float32)
    for l in range(n_layers):
        last = l == n_layers - 1
        agg_lo, agg_hi = _agg_call(h_lo, h_hi, src, dst, an)
        h_lo, h_hi, hxl, hxh = _dense_calls[int(last)](
            agg_lo, agg_hi, h_lo, h_hi, W[l], Wg[l], Ug[l], Wxe[l])
        c0, c1 = _egather_call(hxl.reshape(-1), hxh.reshape(-1), src, dst)
        ep = _edge_call(ep, c0.reshape(E2, DE), c1.reshape(E2, DE),
                        jnp.kron(eye2, We[l]))
    h = jnp.concatenate([h_lo, h_hi], axis=1)
    return h, ep.reshape(E, DE)


def kernel(x, edge_index, A_norm, edge_attr, W, Wg, Ug, We, Wxe):
    src = edge_index[0].astype(jnp.int32)
    dst = edge_index[1].astype(jnp.int32)
    return _run(x, src, dst, A_norm, edge_attr, W, Wg, Ug, We, Wxe)


# speedup vs baseline: 1.2863x; 1.0809x over previous
import functools

import jax
import jax.numpy as jnp
from jax import lax
from jax.experimental import pallas as pl
from jax.experimental.pallas import tpu as pltpu
from jax.experimental.pallas import tpu_sc as plsc

N = 10000
E = 160000
F = 256
DE = 16
FH = F // 2
NC = 2
NS = 16
LANES = 16

EC_AGG = 64
EDGES_PER_TILE = E // NS
NBUF = 3
N_FULL = EDGES_PER_TILE // EC_AGG
N_WAVES = N_FULL // NBUF
TAIL_E = EDGES_PER_TILE - N_FULL * EC_AGG
ROWS_PER_TILE = 624
TAIL_BASE = NS * ROWS_PER_TILE
TAIL_ROWS = N - TAIL_BASE


def _agg_body(h_lo, h_hi, src, dst, an, agg_lo, agg_hi,
              src_v, dst_v, r0, r1, r2, a0, a1, a2, shared,
              g0, g1, g2, q0, q1, q2, ssem):
    c = lax.axis_index("c")
    s = lax.axis_index("s")
    slots = (r0, r1, r2)
    anbs = (a0, a1, a2)
    gsems = (g0, g1, g2)
    asems = (q0, q1, q2)

    ebase = s * EDGES_PER_TILE
    lsrc = pltpu.async_copy(src.at[pl.ds(ebase, EDGES_PER_TILE)], src_v, g0)
    ldst = pltpu.async_copy(dst.at[pl.ds(ebase, EDGES_PER_TILE)], dst_v, g1)

    @pl.loop(0, EC_AGG)
    def _zr(r):
        for j in range(FH // LANES):
            r2[r, pl.ds(j * LANES, LANES)] = jnp.zeros((LANES,), jnp.float32)

    for t in range(ROWS_PER_TILE // EC_AGG):
        pltpu.sync_copy(
            r2, shared.at[pl.ds(s * ROWS_PER_TILE + t * EC_AGG, EC_AGG)])
    rem = ROWS_PER_TILE - (ROWS_PER_TILE // EC_AGG) * EC_AGG
    pltpu.sync_copy(
        r2.at[pl.ds(0, rem)],
        shared.at[pl.ds(s * ROWS_PER_TILE + ROWS_PER_TILE - rem, rem)])

    @pl.when(s == NS - 1)
    def _():
        pltpu.sync_copy(r2.at[pl.ds(0, TAIL_ROWS)],
                        shared.at[pl.ds(TAIL_BASE, TAIL_ROWS)])

    lsrc.wait()
    ldst.wait()
    plsc.subcore_barrier()

    def scale_rows(slot, anb, nrows):
        @pl.loop(0, nrows // 2)
        def _row(r):
            r2 = r * 2
            a0 = plsc.load_gather(anb, [jnp.full((LANES,), r2, jnp.int32)])
            a1 = plsc.load_gather(anb, [jnp.full((LANES,), r2 + 1, jnp.int32)])
            for j in range(FH // LANES):
                sl = pl.ds(j * LANES, LANES)
                slot[r2, sl] = slot[r2, sl] * a0
            for j in range(FH // LANES):
                sl = pl.ds(j * LANES, LANES)
                slot[r2 + 1, sl] = slot[r2 + 1, sl] * a1

    def chunk_loop(tab):
        for b in range(NBUF):
            pltpu.async_copy(
                tab.at[src_v.at[pl.ds(b * EC_AGG, EC_AGG)]],
                slots[b], gsems[b])
            pltpu.async_copy(
                an.at[pl.ds(ebase + b * EC_AGG, EC_AGG)], anbs[b], asems[b])

        @pl.loop(0, N_WAVES)
        def _wave(u):
            for b in range(NBUF):
                i = u * NBUF + b
                cb = i * EC_AGG
                pltpu.make_async_copy(
                    tab.at[src_v.at[pl.ds(0, EC_AGG)]],
                    slots[b], gsems[b]).wait()
                pltpu.make_async_copy(
                    an.at[pl.ds(ebase, EC_AGG)], anbs[b], asems[b]).wait()
                scale_rows(slots[b], anbs[b], EC_AGG)
                sh = []
                for q in range(EC_AGG // LANES):
                    dvec = dst_v[pl.ds(cb + q * LANES, LANES)]
                    sh.append(pltpu.async_copy(
                        slots[b].at[pl.ds(q * LANES, LANES)],
                        shared.at[dvec], ssem, add=True))
                for h in sh:
                    h.wait()

                @pl.when(i + NBUF < N_FULL)
                def _(b=b, i=i):
                    pltpu.async_copy(
                        tab.at[src_v.at[pl.ds((i + NBUF) * EC_AGG, EC_AGG)]],
                        slots[b], gsems[b])
                    pltpu.async_copy(
                        an.at[pl.ds(ebase + (i + NBUF) * EC_AGG, EC_AGG)],
                        anbs[b], asems[b])

        tb = N_FULL * EC_AGG
        pltpu.async_copy(
            tab.at[src_v.at[pl.ds(tb, TAIL_E)]],
            r0.at[pl.ds(0, TAIL_E)], g0).wait()
        pltpu.async_copy(an.at[pl.ds(ebase + tb, TAIL_E)], a0.at[pl.ds(0, TAIL_E)],
                         q0).wait()
        scale_rows(r0, a0, TAIL_E)
        dvec = dst_v[pl.ds(tb, LANES)]
        pltpu.sync_copy(r0.at[pl.ds(0, LANES)], shared.at[dvec], add=True)

    @pl.when(c == 0)
    def _():
        chunk_loop(h_lo)

    @pl.when(c == 1)
    def _():
        chunk_loop(h_hi)

    plsc.subcore_barrier()

    def writeout(out):
        sl = pl.ds(s * ROWS_PER_TILE, ROWS_PER_TILE)
        pltpu.sync_copy(shared.at[sl], out.at[sl])

        @pl.when(s == NS - 1)
        def _():
            tl = pl.ds(TAIL_BASE, TAIL_ROWS)
            pltpu.sync_copy(shared.at[tl], out.at[tl])

    @pl.when(c == 0)
    def _():
        writeout(agg_lo)

    @pl.when(c == 1)
    def _():
        writeout(agg_hi)


_agg_call = pl.kernel(
    _agg_body,
    out_type=(jax.ShapeDtypeStruct((N, FH), jnp.float32),
              jax.ShapeDtypeStruct((N, FH), jnp.float32)),
    mesh=plsc.VectorSubcoreMesh(core_axis_name="c", subcore_axis_name="s"),
    compiler_params=pltpu.CompilerParams(needs_layout_passes=False),
    scratch_types=[
        pltpu.VMEM((EDGES_PER_TILE,), jnp.int32),
        pltpu.VMEM((EDGES_PER_TILE,), jnp.int32),
        pltpu.VMEM((EC_AGG, FH), jnp.float32),
        pltpu.VMEM((EC_AGG, FH), jnp.float32),
        pltpu.VMEM((EC_AGG, FH), jnp.float32),
        pltpu.VMEM((EC_AGG,), jnp.float32),
        pltpu.VMEM((EC_AGG,), jnp.float32),
        pltpu.VMEM((EC_AGG,), jnp.float32),
        pltpu.VMEM_SHARED((N, FH), jnp.float32),
        pltpu.SemaphoreType.DMA,
        pltpu.SemaphoreType.DMA,
        pltpu.SemaphoreType.DMA,
        pltpu.SemaphoreType.DMA,
        pltpu.SemaphoreType.DMA,
        pltpu.SemaphoreType.DMA,
        pltpu.SemaphoreType.DMA,
    ],
)

DH = DE // 2
E2 = E // 2
EDGES_PER_SUB = E // NS
G_BLK = 2000
G_NBLK = EDGES_PER_SUB // G_BLK
G_GROUPS = G_BLK // LANES


def _egather_body(hxl, hxh, src, dst, out0, out1, src_v, dst_v, tab_v, ob_v):
    c = lax.axis_index("c")
    s = lax.axis_index("s")
    iota = lax.iota(jnp.int32, 16)
    base16 = lax.shift_right_logical(iota, 3)
    col = lax.bitwise_and(iota, 7)

    pltpu.sync_copy(src.at[pl.ds(s * EDGES_PER_SUB, EDGES_PER_SUB)], src_v)
    pltpu.sync_copy(dst.at[pl.ds(s * EDGES_PER_SUB, EDGES_PER_SUB)], dst_v)

    def run(tab, out):
        pltpu.sync_copy(tab, tab_v)

        @pl.loop(0, G_NBLK)
        def _blk(b):
            @pl.loop(0, G_GROUPS)
            def _grp(g):
                eb = (b * G_GROUPS + g) * LANES
                for j in range(LANES // 2):
                    idx = base16 + (eb + 2 * j)
                    rows_s = plsc.load_gather(src_v, [idx])
                    rows_d = plsc.load_gather(dst_v, [idx])
                    fs = lax.bitwise_or(lax.shift_left(rows_s, 3), col)
                    fd = lax.bitwise_or(lax.shift_left(rows_d, 3), col)
                    vs = plsc.load_gather(tab_v, [fs])
                    vd = plsc.load_gather(tab_v, [fd])
                    ob_v[pl.ds((g * (LANES // 2) + j) * LANES, LANES)] = vs + vd

            pltpu.sync_copy(
                ob_v, out.at[pl.ds((s * EDGES_PER_SUB + b * G_BLK) * DH,
                                   G_BLK * DH)])

    @pl.when(c == 0)
    def _():
        run(hxl, out0)

    @pl.when(c == 1)
    def _():
        run(hxh, out1)


_egather_call = pl.kernel(
    _egather_body,
    out_type=(jax.ShapeDtypeStruct((E * DH,), jnp.float32),
              jax.ShapeDtypeStruct((E * DH,), jnp.float32)),
    mesh=plsc.VectorSubcoreMesh(core_axis_name="c", subcore_axis_name="s"),
    compiler_params=pltpu.CompilerParams(needs_layout_passes=False),
    scratch_types=[
        pltpu.VMEM((EDGES_PER_SUB,), jnp.int32),
        pltpu.VMEM((EDGES_PER_SUB,), jnp.int32),
        pltpu.VMEM((N * DH,), jnp.float32),
        pltpu.VMEM((G_BLK * DH,), jnp.float32),
    ],
)

RB = 1000


def _dense_body(last, agg_lo_ref, agg_hi_ref, h_lo_ref, h_hi_ref,
                w_ref, wg_ref, ug_ref, wxe_ref,
                hn_lo_ref, hn_hi_ref, hxl_ref, hxh_ref):
    h = jnp.concatenate([h_lo_ref[...], h_hi_ref[...]], axis=1)
    agg = jnp.concatenate([agg_lo_ref[...], agg_hi_ref[...]], axis=1)
    hl = jnp.dot(agg, w_ref[...], preferred_element_type=jnp.float32)
    g = jax.nn.sigmoid(
        jnp.dot(h, wg_ref[...], preferred_element_type=jnp.float32)
        + jnp.dot(hl, ug_ref[...], preferred_element_type=jnp.float32))
    hn = g * hl + (1.0 - g) * h
    mu = jnp.mean(hn, axis=-1, keepdims=True)
    var = jnp.mean((hn - mu) * (hn - mu), axis=-1, keepdims=True)
    hn = (hn - mu) * lax.rsqrt(var + 1e-5)
    hx = jnp.dot(hn, wxe_ref[...], preferred_element_type=jnp.float32)
    hxl_ref[...] = hx[:, :DH]
    hxh_ref[...] = hx[:, DH:]
    if last:
        hn = jnp.maximum(hn, 0.0)
    hn_lo_ref[...] = hn[:, :FH]
    hn_hi_ref[...] = hn[:, FH:]


def _make_dense(last):
    return pl.pallas_call(
        functools.partial(_dense_body, last),
        grid=(N // RB,),
        in_specs=[
            pl.BlockSpec((RB, FH), lambda i: (i, 0)),
            pl.BlockSpec((RB, FH), lambda i: (i, 0)),
            pl.BlockSpec((RB, FH), lambda i: (i, 0)),
            pl.BlockSpec((RB, FH), lambda i: (i, 0)),
            pl.BlockSpec((F, F), lambda i: (0, 0)),
            pl.BlockSpec((F, F), lambda i: (0, 0)),
            pl.BlockSpec((F, F), lambda i: (0, 0)),
            pl.BlockSpec((F, DE), lambda i: (0, 0)),
        ],
        out_specs=(
            pl.BlockSpec((RB, FH), lambda i: (i, 0)),
            pl.BlockSpec((RB, FH), lambda i: (i, 0)),
            pl.BlockSpec((RB, DH), lambda i: (i, 0)),
            pl.BlockSpec((RB, DH), lambda i: (i, 0)),
        ),
        out_shape=(
            jax.ShapeDtypeStruct((N, FH), jnp.float32),
            jax.ShapeDtypeStruct((N, FH), jnp.float32),
            jax.ShapeDtypeStruct((N, DH), jnp.float32),
            jax.ShapeDtypeStruct((N, DH), jnp.float32),
        ),
    )


_dense_calls = (_make_dense(False), _make_dense(True))

EB2 = 8000


def _edge_body(e_ref, c0_ref, c1_ref, we_ref, out_ref):
    c0 = c0_ref[...]
    c1 = c1_ref[...]
    contrib = jnp.concatenate(
        [c0[:, :DH], c1[:, :DH], c0[:, DH:], c1[:, DH:]], axis=1)
    out_ref[...] = jnp.maximum(
        jnp.dot(e_ref[...], we_ref[...], preferred_element_type=jnp.float32)
        + contrib, 0.0)


_edge_call = pl.pallas_call(
    _edge_body,
    grid=(E2 // EB2,),
    in_specs=[
        pl.BlockSpec((EB2, 2 * DE), lambda i: (i, 0)),
        pl.BlockSpec((EB2, DE), lambda i: (i, 0)),
        pl.BlockSpec((EB2, DE), lambda i: (i, 0)),
        pl.BlockSpec((2 * DE, 2 * DE), lambda i: (0, 0)),
    ],
    out_specs=pl.BlockSpec((EB2, 2 * DE), lambda i: (i, 0)),
    out_shape=jax.ShapeDtypeStruct((E2, 2 * DE), jnp.float32),
)


@jax.jit
def _run(x, src, dst, an, e0, W, Wg, Ug, We, Wxe):
    n_layers = W.shape[0]
    h_lo = x[:, :FH]
    h_hi = x[:, FH:]
    ep = e0.reshape(E2, 2 * DE)
    eye2 = jnp.eye(2, dtype=jnp.float32)
    for l in range(n_layers):
        last = l == n_layers - 1
        agg_lo, agg_hi = _agg_call(h_lo, h_hi, src, dst, an)
        h_lo, h_hi, hxl, hxh = _dense_calls[int(last)](
            agg_lo, agg_hi, h_lo, h_hi, W[l], Wg[l], Ug[l], Wxe[l])
        c0, c1 = _egather_call(hxl.reshape(-1), hxh.reshape(-1), src, dst)
        ep = _edge_call(ep, c0.reshape(E2, DE), c1.reshape(E2, DE),
                        jnp.kron(eye2, We[l]))
    h = jnp.concatenate([h_lo, h_hi], axis=1)
    return h, ep.reshape(E, DE)


def kernel(x, edge_index, A_norm, edge_attr, W, Wg, Ug, We, Wxe):
    src = edge_index[0].astype(jnp.int32)
    dst = edge_index[1].astype(jnp.int32)
    return _run(x, src, dst, A_norm, edge_attr, W, Wg, Ug, We, Wxe)


# speedup vs baseline: 1.3267x; 1.0314x over previous
import functools

import jax
import jax.numpy as jnp
from jax import lax
from jax.experimental import pallas as pl
from jax.experimental.pallas import tpu as pltpu
from jax.experimental.pallas import tpu_sc as plsc

N = 10000
E = 160000
F = 256
DE = 16
FH = F // 2
NC = 2
NS = 16
LANES = 16

EC_AGG = 64
EDGES_PER_TILE = E // NS
NBUF = 3
N_FULL = EDGES_PER_TILE // EC_AGG
N_WAVES = N_FULL // NBUF
TAIL_E = EDGES_PER_TILE - N_FULL * EC_AGG
ROWS_PER_TILE = 624
TAIL_BASE = NS * ROWS_PER_TILE
TAIL_ROWS = N - TAIL_BASE


def _agg_body(h_lo, h_hi, src, dst, an, agg_lo, agg_hi,
              src_v, dst_v, r0, r1, r2, a0, a1, a2, shared,
              g0, g1, g2, q0, q1, q2, ssem):
    c = lax.axis_index("c")
    s = lax.axis_index("s")
    slots = (r0, r1, r2)
    anbs = (a0, a1, a2)
    gsems = (g0, g1, g2)
    asems = (q0, q1, q2)

    ebase = s * EDGES_PER_TILE
    lsrc = pltpu.async_copy(src.at[pl.ds(ebase, EDGES_PER_TILE)], src_v, g0)
    ldst = pltpu.async_copy(dst.at[pl.ds(ebase, EDGES_PER_TILE)], dst_v, g1)

    @pl.loop(0, EC_AGG)
    def _zr(r):
        for j in range(FH // LANES):
            r2[r, pl.ds(j * LANES, LANES)] = jnp.zeros((LANES,), jnp.float32)

    for t in range(ROWS_PER_TILE // EC_AGG):
        pltpu.sync_copy(
            r2, shared.at[pl.ds(s * ROWS_PER_TILE + t * EC_AGG, EC_AGG)])
    rem = ROWS_PER_TILE - (ROWS_PER_TILE // EC_AGG) * EC_AGG
    pltpu.sync_copy(
        r2.at[pl.ds(0, rem)],
        shared.at[pl.ds(s * ROWS_PER_TILE + ROWS_PER_TILE - rem, rem)])

    @pl.when(s == NS - 1)
    def _():
        pltpu.sync_copy(r2.at[pl.ds(0, TAIL_ROWS)],
                        shared.at[pl.ds(TAIL_BASE, TAIL_ROWS)])

    lsrc.wait()
    ldst.wait()
    plsc.subcore_barrier()

    def scale_rows(slot, anb, nrows):
        @pl.loop(0, nrows // 4)
        def _row(r):
            r4 = r * 4
            avs = [plsc.load_gather(anb, [jnp.full((LANES,), r4 + k, jnp.int32)])
                   for k in range(4)]
            for k in range(4):
                for j in range(FH // LANES):
                    sl = pl.ds(j * LANES, LANES)
                    slot[r4 + k, sl] = slot[r4 + k, sl] * avs[k]

    def chunk_loop(tab):
        for b in range(NBUF):
            pltpu.async_copy(
                tab.at[src_v.at[pl.ds(b * EC_AGG, EC_AGG)]],
                slots[b], gsems[b])
            pltpu.async_copy(
                an.at[pl.ds(ebase + b * EC_AGG, EC_AGG)], anbs[b], asems[b])

        @pl.loop(0, N_WAVES)
        def _wave(u):
            for b in range(NBUF):
                i = u * NBUF + b
                cb = i * EC_AGG
                pltpu.make_async_copy(
                    tab.at[src_v.at[pl.ds(0, EC_AGG)]],
                    slots[b], gsems[b]).wait()
                pltpu.make_async_copy(
                    an.at[pl.ds(ebase, EC_AGG)], anbs[b], asems[b]).wait()
                scale_rows(slots[b], anbs[b], EC_AGG)
                sh = []
                for q in range(EC_AGG // LANES):
                    dvec = dst_v[pl.ds(cb + q * LANES, LANES)]
                    sh.append(pltpu.async_copy(
                        slots[b].at[pl.ds(q * LANES, LANES)],
                        shared.at[dvec], ssem, add=True))
                for h in sh:
                    h.wait()

                @pl.when(i + NBUF < N_FULL)
                def _(b=b, i=i):
                    pltpu.async_copy(
                        tab.at[src_v.at[pl.ds((i + NBUF) * EC_AGG, EC_AGG)]],
                        slots[b], gsems[b])
                    pltpu.async_copy(
                        an.at[pl.ds(ebase + (i + NBUF) * EC_AGG, EC_AGG)],
                        anbs[b], asems[b])

        tb = N_FULL * EC_AGG
        pltpu.async_copy(
            tab.at[src_v.at[pl.ds(tb, TAIL_E)]],
            r0.at[pl.ds(0, TAIL_E)], g0).wait()
        pltpu.async_copy(an.at[pl.ds(ebase + tb, TAIL_E)], a0.at[pl.ds(0, TAIL_E)],
                         q0).wait()
        scale_rows(r0, a0, TAIL_E)
        dvec = dst_v[pl.ds(tb, LANES)]
        pltpu.sync_copy(r0.at[pl.ds(0, LANES)], shared.at[dvec], add=True)

    @pl.when(c == 0)
    def _():
        chunk_loop(h_lo)

    @pl.when(c == 1)
    def _():
        chunk_loop(h_hi)

    plsc.subcore_barrier()

    def writeout(out):
        sl = pl.ds(s * ROWS_PER_TILE, ROWS_PER_TILE)
        pltpu.sync_copy(shared.at[sl], out.at[sl])

        @pl.when(s == NS - 1)
        def _():
            tl = pl.ds(TAIL_BASE, TAIL_ROWS)
            pltpu.sync_copy(shared.at[tl], out.at[tl])

    @pl.when(c == 0)
    def _():
        writeout(agg_lo)

    @pl.when(c == 1)
    def _():
        writeout(agg_hi)


_agg_call = pl.kernel(
    _agg_body,
    out_type=(jax.ShapeDtypeStruct((N, FH), jnp.float32),
              jax.ShapeDtypeStruct((N, FH), jnp.float32)),
    mesh=plsc.VectorSubcoreMesh(core_axis_name="c", subcore_axis_name="s"),
    compiler_params=pltpu.CompilerParams(needs_layout_passes=False),
    scratch_types=[
        pltpu.VMEM((EDGES_PER_TILE,), jnp.int32),
        pltpu.VMEM((EDGES_PER_TILE,), jnp.int32),
        pltpu.VMEM((EC_AGG, FH), jnp.float32),
        pltpu.VMEM((EC_AGG, FH), jnp.float32),
        pltpu.VMEM((EC_AGG, FH), jnp.float32),
        pltpu.VMEM((EC_AGG,), jnp.float32),
        pltpu.VMEM((EC_AGG,), jnp.float32),
        pltpu.VMEM((EC_AGG,), jnp.float32),
        pltpu.VMEM_SHARED((N, FH), jnp.float32),
        pltpu.SemaphoreType.DMA,
        pltpu.SemaphoreType.DMA,
        pltpu.SemaphoreType.DMA,
        pltpu.SemaphoreType.DMA,
        pltpu.SemaphoreType.DMA,
        pltpu.SemaphoreType.DMA,
        pltpu.SemaphoreType.DMA,
    ],
)

DH = DE // 2
E2 = E // 2
EDGES_PER_SUB = E // NS
G_BLK = 2000
G_NBLK = EDGES_PER_SUB // G_BLK
G_GROUPS = G_BLK // LANES


def _egather_body(hxl, hxh, src, dst, out0, out1, src_v, dst_v, tab_v, ob_v):
    c = lax.axis_index("c")
    s = lax.axis_index("s")
    iota = lax.iota(jnp.int32, 16)
    base16 = lax.shift_right_logical(iota, 3)
    col = lax.bitwise_and(iota, 7)

    pltpu.sync_copy(src.at[pl.ds(s * EDGES_PER_SUB, EDGES_PER_SUB)], src_v)
    pltpu.sync_copy(dst.at[pl.ds(s * EDGES_PER_SUB, EDGES_PER_SUB)], dst_v)

    def run(tab, out):
        pltpu.sync_copy(tab, tab_v)

        @pl.loop(0, G_NBLK)
        def _blk(b):
            @pl.loop(0, G_GROUPS)
            def _grp(g):
                eb = (b * G_GROUPS + g) * LANES
                for j in range(LANES // 2):
                    idx = base16 + (eb + 2 * j)
                    rows_s = plsc.load_gather(src_v, [idx])
                    rows_d = plsc.load_gather(dst_v, [idx])
                    fs = lax.bitwise_or(lax.shift_left(rows_s, 3), col)
                    fd = lax.bitwise_or(lax.shift_left(rows_d, 3), col)
                    vs = plsc.load_gather(tab_v, [fs])
                    vd = plsc.load_gather(tab_v, [fd])
                    ob_v[pl.ds((g * (LANES // 2) + j) * LANES, LANES)] = vs + vd

            pltpu.sync_copy(
                ob_v, out.at[pl.ds((s * EDGES_PER_SUB + b * G_BLK) * DH,
                                   G_BLK * DH)])

    @pl.when(c == 0)
    def _():
        run(hxl, out0)

    @pl.when(c == 1)
    def _():
        run(hxh, out1)


_egather_call = pl.kernel(
    _egather_body,
    out_type=(jax.ShapeDtypeStruct((E * DH,), jnp.float32),
              jax.ShapeDtypeStruct((E * DH,), jnp.float32)),
    mesh=plsc.VectorSubcoreMesh(core_axis_name="c", subcore_axis_name="s"),
    compiler_params=pltpu.CompilerParams(needs_layout_passes=False),
    scratch_types=[
        pltpu.VMEM((EDGES_PER_SUB,), jnp.int32),
        pltpu.VMEM((EDGES_PER_SUB,), jnp.int32),
        pltpu.VMEM((N * DH,), jnp.float32),
        pltpu.VMEM((G_BLK * DH,), jnp.float32),
    ],
)

RB = 1000


def _dense_body(last, agg_lo_ref, agg_hi_ref, h_lo_ref, h_hi_ref,
                w_ref, wg_ref, ug_ref, wxe_ref,
                hn_lo_ref, hn_hi_ref, hxl_ref, hxh_ref):
    h = jnp.concatenate([h_lo_ref[...], h_hi_ref[...]], axis=1)
    agg = jnp.concatenate([agg_lo_ref[...], agg_hi_ref[...]], axis=1)
    hl = jnp.dot(agg, w_ref[...], preferred_element_type=jnp.float32)
    g = jax.nn.sigmoid(
        jnp.dot(h, wg_ref[...], preferred_element_type=jnp.float32)
        + jnp.dot(hl, ug_ref[...], preferred_element_type=jnp.float32))
    hn = g * hl + (1.0 - g) * h
    mu = jnp.mean(hn, axis=-1, keepdims=True)
    var = jnp.mean((hn - mu) * (hn - mu), axis=-1, keepdims=True)
    hn = (hn - mu) * lax.rsqrt(var + 1e-5)
    hx = jnp.dot(hn, wxe_ref[...], preferred_element_type=jnp.float32)
    hxl_ref[...] = hx[:, :DH]
    hxh_ref[...] = hx[:, DH:]
    if last:
        hn = jnp.maximum(hn, 0.0)
    hn_lo_ref[...] = hn[:, :FH]
    hn_hi_ref[...] = hn[:, FH:]


def _make_dense(last):
    return pl.pallas_call(
        functools.partial(_dense_body, last),
        grid=(N // RB,),
        in_specs=[
            pl.BlockSpec((RB, FH), lambda i: (i, 0)),
            pl.BlockSpec((RB, FH), lambda i: (i, 0)),
            pl.BlockSpec((RB, FH), lambda i: (i, 0)),
            pl.BlockSpec((RB, FH), lambda i: (i, 0)),
            pl.BlockSpec((F, F), lambda i: (0, 0)),
            pl.BlockSpec((F, F), lambda i: (0, 0)),
            pl.BlockSpec((F, F), lambda i: (0, 0)),
            pl.BlockSpec((F, DE), lambda i: (0, 0)),
        ],
        out_specs=(
            pl.BlockSpec((RB, FH), lambda i: (i, 0)),
            pl.BlockSpec((RB, FH), lambda i: (i, 0)),
            pl.BlockSpec((RB, DH), lambda i: (i, 0)),
            pl.BlockSpec((RB, DH), lambda i: (i, 0)),
        ),
        out_shape=(
            jax.ShapeDtypeStruct((N, FH), jnp.float32),
            jax.ShapeDtypeStruct((N, FH), jnp.float32),
            jax.ShapeDtypeStruct((N, DH), jnp.float32),
            jax.ShapeDtypeStruct((N, DH), jnp.float32),
        ),
    )


_dense_calls = (_make_dense(False), _make_dense(True))

EB2 = 8000


def _edge_body(e_ref, c0_ref, c1_ref, we_ref, out_ref):
    c0 = c0_ref[...]
    c1 = c1_ref[...]
    contrib = jnp.concatenate(
        [c0[:, :DH], c1[:, :DH], c0[:, DH:], c1[:, DH:]], axis=1)
    out_ref[...] = jnp.maximum(
        jnp.dot(e_ref[...], we_ref[...], preferred_element_type=jnp.float32)
        + contrib, 0.0)


_edge_call = pl.pallas_call(
    _edge_body,
    grid=(E2 // EB2,),
    in_specs=[
        pl.BlockSpec((EB2, 2 * DE), lambda i: (i, 0)),
        pl.BlockSpec((EB2, DE), lambda i: (i, 0)),
        pl.BlockSpec((EB2, DE), lambda i: (i, 0)),
        pl.BlockSpec((2 * DE, 2 * DE), lambda i: (0, 0)),
    ],
    out_specs=pl.BlockSpec((EB2, 2 * DE), lambda i: (i, 0)),
    out_shape=jax.ShapeDtypeStruct((E2, 2 * DE), jnp.float32),
)


@jax.jit
def _run(x, src, dst, an, e0, W, Wg, Ug, We, Wxe):
    n_layers = W.shape[0]
    h_lo = x[:, :FH]
    h_hi = x[:, FH:]
    ep = e0.reshape(E2, 2 * DE)
    eye2 = jnp.eye(2, dtype=jnp.float32)
    for l in range(n_layers):
        last = l == n_layers - 1
        agg_lo, agg_hi = _agg_call(h_lo, h_hi, src, dst, an)
        h_lo, h_hi, hxl, hxh = _dense_calls[int(last)](
            agg_lo, agg_hi, h_lo, h_hi, W[l], Wg[l], Ug[l], Wxe[l])
        c0, c1 = _egather_call(hxl.reshape(-1), hxh.reshape(-1), src, dst)
        ep = _edge_call(ep, c0.reshape(E2, DE), c1.reshape(E2, DE),
                        jnp.kron(eye2, We[l]))
    h = jnp.concatenate([h_lo, h_hi], axis=1)
    return h, ep.reshape(E, DE)


def kernel(x, edge_index, A_norm, edge_attr, W, Wg, Ug, We, Wxe):
    src = edge_index[0].astype(jnp.int32)
    dst = edge_index[1].astype(jnp.int32)
    return _run(x, src, dst, A_norm, edge_attr, W, Wg, Ug, We, Wxe)


# speedup vs baseline: 1.3369x; 1.0077x over previous
import functools

import jax
import jax.numpy as jnp
from jax import lax
from jax.experimental import pallas as pl
from jax.experimental.pallas import tpu as pltpu
from jax.experimental.pallas import tpu_sc as plsc

N = 10000
E = 160000
F = 256
DE = 16
FH = F // 2
NC = 2
NS = 16
LANES = 16

EC_AGG = 64
EDGES_PER_TILE = E // NS
NBUF = 3
N_FULL = EDGES_PER_TILE // EC_AGG
N_WAVES = N_FULL // NBUF
TAIL_E = EDGES_PER_TILE - N_FULL * EC_AGG
ROWS_PER_TILE = 624
TAIL_BASE = NS * ROWS_PER_TILE
TAIL_ROWS = N - TAIL_BASE


def _agg_body(h_lo, h_hi, src, dst, an, agg_lo, agg_hi,
              src_v, dst_v, r0, r1, r2, a0, a1, a2, shared,
              g0, g1, g2, q0, q1, q2, ssem):
    c = lax.axis_index("c")
    s = lax.axis_index("s")
    slots = (r0, r1, r2)
    anbs = (a0, a1, a2)
    gsems = (g0, g1, g2)
    asems = (q0, q1, q2)

    ebase = s * EDGES_PER_TILE
    lsrc = pltpu.async_copy(src.at[pl.ds(ebase, EDGES_PER_TILE)], src_v, g0)
    ldst = pltpu.async_copy(dst.at[pl.ds(ebase, EDGES_PER_TILE)], dst_v, g1)

    @pl.loop(0, EC_AGG)
    def _zr(r):
        for j in range(FH // LANES):
            r2[r, pl.ds(j * LANES, LANES)] = jnp.zeros((LANES,), jnp.float32)

    for t in range(ROWS_PER_TILE // EC_AGG):
        pltpu.sync_copy(
            r2, shared.at[pl.ds(s * ROWS_PER_TILE + t * EC_AGG, EC_AGG)])
    rem = ROWS_PER_TILE - (ROWS_PER_TILE // EC_AGG) * EC_AGG
    pltpu.sync_copy(
        r2.at[pl.ds(0, rem)],
        shared.at[pl.ds(s * ROWS_PER_TILE + ROWS_PER_TILE - rem, rem)])

    @pl.when(s == NS - 1)
    def _():
        pltpu.sync_copy(r2.at[pl.ds(0, TAIL_ROWS)],
                        shared.at[pl.ds(TAIL_BASE, TAIL_ROWS)])

    lsrc.wait()
    ldst.wait()
    plsc.subcore_barrier()

    def scale_rows(slot, anb, nrows):
        @pl.loop(0, nrows // 8)
        def _row(r):
            r8 = r * 8
            avs = [plsc.load_gather(anb, [jnp.full((LANES,), r8 + k, jnp.int32)])
                   for k in range(8)]
            for k in range(8):
                for j in range(FH // LANES):
                    sl = pl.ds(j * LANES, LANES)
                    slot[r8 + k, sl] = slot[r8 + k, sl] * avs[k]

    def chunk_loop(tab):
        for b in range(NBUF):
            pltpu.async_copy(
                tab.at[src_v.at[pl.ds(b * EC_AGG, EC_AGG)]],
                slots[b], gsems[b])
            pltpu.async_copy(
                an.at[pl.ds(ebase + b * EC_AGG, EC_AGG)], anbs[b], asems[b])

        @pl.loop(0, N_WAVES)
        def _wave(u):
            for b in range(NBUF):
                i = u * NBUF + b
                cb = i * EC_AGG
                pltpu.make_async_copy(
                    tab.at[src_v.at[pl.ds(0, EC_AGG)]],
                    slots[b], gsems[b]).wait()
                pltpu.make_async_copy(
                    an.at[pl.ds(ebase, EC_AGG)], anbs[b], asems[b]).wait()
                scale_rows(slots[b], anbs[b], EC_AGG)
                sh = []
                for q in range(EC_AGG // LANES):
                    dvec = dst_v[pl.ds(cb + q * LANES, LANES)]
                    sh.append(pltpu.async_copy(
                        slots[b].at[pl.ds(q * LANES, LANES)],
                        shared.at[dvec], ssem, add=True))
                for h in sh:
                    h.wait()

                @pl.when(i + NBUF < N_FULL)
                def _(b=b, i=i):
                    pltpu.async_copy(
                        tab.at[src_v.at[pl.ds((i + NBUF) * EC_AGG, EC_AGG)]],
                        slots[b], gsems[b])
                    pltpu.async_copy(
                        an.at[pl.ds(ebase + (i + NBUF) * EC_AGG, EC_AGG)],
                        anbs[b], asems[b])

        tb = N_FULL * EC_AGG
        pltpu.async_copy(
            tab.at[src_v.at[pl.ds(tb, TAIL_E)]],
            r0.at[pl.ds(0, TAIL_E)], g0).wait()
        pltpu.async_copy(an.at[pl.ds(ebase + tb, TAIL_E)], a0.at[pl.ds(0, TAIL_E)],
                         q0).wait()
        scale_rows(r0, a0, TAIL_E)
        dvec = dst_v[pl.ds(tb, LANES)]
        pltpu.sync_copy(r0.at[pl.ds(0, LANES)], shared.at[dvec], add=True)

    @pl.when(c == 0)
    def _():
        chunk_loop(h_lo)

    @pl.when(c == 1)
    def _():
        chunk_loop(h_hi)

    plsc.subcore_barrier()

    def writeout(out):
        sl = pl.ds(s * ROWS_PER_TILE, ROWS_PER_TILE)
        pltpu.sync_copy(shared.at[sl], out.at[sl])

        @pl.when(s == NS - 1)
        def _():
            tl = pl.ds(TAIL_BASE, TAIL_ROWS)
            pltpu.sync_copy(shared.at[tl], out.at[tl])

    @pl.when(c == 0)
    def _():
        writeout(agg_lo)

    @pl.when(c == 1)
    def _():
        writeout(agg_hi)


_agg_call = pl.kernel(
    _agg_body,
    out_type=(jax.ShapeDtypeStruct((N, FH), jnp.float32),
              jax.ShapeDtypeStruct((N, FH), jnp.float32)),
    mesh=plsc.VectorSubcoreMesh(core_axis_name="c", subcore_axis_name="s"),
    compiler_params=pltpu.CompilerParams(needs_layout_passes=False),
    scratch_types=[
        pltpu.VMEM((EDGES_PER_TILE,), jnp.int32),
        pltpu.VMEM((EDGES_PER_TILE,), jnp.int32),
        pltpu.VMEM((EC_AGG, FH), jnp.float32),
        pltpu.VMEM((EC_AGG, FH), jnp.float32),
        pltpu.VMEM((EC_AGG, FH), jnp.float32),
        pltpu.VMEM((EC_AGG,), jnp.float32),
        pltpu.VMEM((EC_AGG,), jnp.float32),
        pltpu.VMEM((EC_AGG,), jnp.float32),
        pltpu.VMEM_SHARED((N, FH), jnp.float32),
        pltpu.SemaphoreType.DMA,
        pltpu.SemaphoreType.DMA,
        pltpu.SemaphoreType.DMA,
        pltpu.SemaphoreType.DMA,
        pltpu.SemaphoreType.DMA,
        pltpu.SemaphoreType.DMA,
        pltpu.SemaphoreType.DMA,
    ],
)

DH = DE // 2
E2 = E // 2
EDGES_PER_SUB = E // NS
G_BLK = 2000
G_NBLK = EDGES_PER_SUB // G_BLK
G_GROUPS = G_BLK // LANES


def _egather_body(hxl, hxh, src, dst, out0, out1, src_v, dst_v, tab_v, ob_v):
    c = lax.axis_index("c")
    s = lax.axis_index("s")
    iota = lax.iota(jnp.int32, 16)
    base16 = lax.shift_right_logical(iota, 3)
    col = lax.bitwise_and(iota, 7)

    pltpu.sync_copy(src.at[pl.ds(s * EDGES_PER_SUB, EDGES_PER_SUB)], src_v)
    pltpu.sync_copy(dst.at[pl.ds(s * EDGES_PER_SUB, EDGES_PER_SUB)], dst_v)

    def run(tab, out):
        pltpu.sync_copy(tab, tab_v)

        @pl.loop(0, G_NBLK)
        def _blk(b):
            @pl.loop(0, G_GROUPS)
            def _grp(g):
                eb = (b * G_GROUPS + g) * LANES
                for j in range(LANES // 2):
                    idx = base16 + (eb + 2 * j)
                    rows_s = plsc.load_gather(src_v, [idx])
                    rows_d = plsc.load_gather(dst_v, [idx])
                    fs = lax.bitwise_or(lax.shift_left(rows_s, 3), col)
                    fd = lax.bitwise_or(lax.shift_left(rows_d, 3), col)
                    vs = plsc.load_gather(tab_v, [fs])
                    vd = plsc.load_gather(tab_v, [fd])
                    ob_v[pl.ds((g * (LANES // 2) + j) * LANES, LANES)] = vs + vd

            pltpu.sync_copy(
                ob_v, out.at[pl.ds((s * EDGES_PER_SUB + b * G_BLK) * DH,
                                   G_BLK * DH)])

    @pl.when(c == 0)
    def _():
        run(hxl, out0)

    @pl.when(c == 1)
    def _():
        run(hxh, out1)


_egather_call = pl.kernel(
    _egather_body,
    out_type=(jax.ShapeDtypeStruct((E * DH,), jnp.float32),
              jax.ShapeDtypeStruct((E * DH,), jnp.float32)),
    mesh=plsc.VectorSubcoreMesh(core_axis_name="c", subcore_axis_name="s"),
    compiler_params=pltpu.CompilerParams(needs_layout_passes=False),
    scratch_types=[
        pltpu.VMEM((EDGES_PER_SUB,), jnp.int32),
        pltpu.VMEM((EDGES_PER_SUB,), jnp.int32),
        pltpu.VMEM((N * DH,), jnp.float32),
        pltpu.VMEM((G_BLK * DH,), jnp.float32),
    ],
)

RB = 1000


def _dense_body(last, agg_lo_ref, agg_hi_ref, h_lo_ref, h_hi_ref,
                w_ref, wg_ref, ug_ref, wxe_ref,
                hn_lo_ref, hn_hi_ref, hxl_ref, hxh_ref):
    h = jnp.concatenate([h_lo_ref[...], h_hi_ref[...]], axis=1)
    agg = jnp.concatenate([agg_lo_ref[...], agg_hi_ref[...]], axis=1)
    hl = jnp.dot(agg, w_ref[...], preferred_element_type=jnp.float32)
    g = jax.nn.sigmoid(
        jnp.dot(h, wg_ref[...], preferred_element_type=jnp.float32)
        + jnp.dot(hl, ug_ref[...], preferred_element_type=jnp.float32))
    hn = g * hl + (1.0 - g) * h
    mu = jnp.mean(hn, axis=-1, keepdims=True)
    var = jnp.mean((hn - mu) * (hn - mu), axis=-1, keepdims=True)
    hn = (hn - mu) * lax.rsqrt(var + 1e-5)
    hx = jnp.dot(hn, wxe_ref[...], preferred_element_type=jnp.float32)
    hxl_ref[...] = hx[:, :DH]
    hxh_ref[...] = hx[:, DH:]
    if last:
        hn = jnp.maximum(hn, 0.0)
    hn_lo_ref[...] = hn[:, :FH]
    hn_hi_ref[...] = hn[:, FH:]


def _make_dense(last):
    return pl.pallas_call(
        functools.partial(_dense_body, last),
        grid=(N // RB,),
        in_specs=[
            pl.BlockSpec((RB, FH), lambda i: (i, 0)),
            pl.BlockSpec((RB, FH), lambda i: (i, 0)),
            pl.BlockSpec((RB, FH), lambda i: (i, 0)),
            pl.BlockSpec((RB, FH), lambda i: (i, 0)),
            pl.BlockSpec((F, F), lambda i: (0, 0)),
            pl.BlockSpec((F, F), lambda i: (0, 0)),
            pl.BlockSpec((F, F), lambda i: (0, 0)),
            pl.BlockSpec((F, DE), lambda i: (0, 0)),
        ],
        out_specs=(
            pl.BlockSpec((RB, FH), lambda i: (i, 0)),
            pl.BlockSpec((RB, FH), lambda i: (i, 0)),
            pl.BlockSpec((RB, DH), lambda i: (i, 0)),
            pl.BlockSpec((RB, DH), lambda i: (i, 0)),
        ),
        out_shape=(
            jax.ShapeDtypeStruct((N, FH), jnp.float32),
            jax.ShapeDtypeStruct((N, FH), jnp.float32),
            jax.ShapeDtypeStruct((N, DH), jnp.float32),
            jax.ShapeDtypeStruct((N, DH), jnp.float32),
        ),
    )


_dense_calls = (_make_dense(False), _make_dense(True))

EB2 = 8000


def _edge_body(e_ref, c0_ref, c1_ref, we_ref, out_ref):
    c0 = c0_ref[...]
    c1 = c1_ref[...]
    contrib = jnp.concatenate(
        [c0[:, :DH], c1[:, :DH], c0[:, DH:], c1[:, DH:]], axis=1)
    out_ref[...] = jnp.maximum(
        jnp.dot(e_ref[...], we_ref[...], preferred_element_type=jnp.float32)
        + contrib, 0.0)


_edge_call = pl.pallas_call(
    _edge_body,
    grid=(E2 // EB2,),
    in_specs=[
        pl.BlockSpec((EB2, 2 * DE), lambda i: (i, 0)),
        pl.BlockSpec((EB2, DE), lambda i: (i, 0)),
        pl.BlockSpec((EB2, DE), lambda i: (i, 0)),
        pl.BlockSpec((2 * DE, 2 * DE), lambda i: (0, 0)),
    ],
    out_specs=pl.BlockSpec((EB2, 2 * DE), lambda i: (i, 0)),
    out_shape=jax.ShapeDtypeStruct((E2, 2 * DE), jnp.float32),
)


@jax.jit
def _run(x, src, dst, an, e0, W, Wg, Ug, We, Wxe):
    n_layers = W.shape[0]
    h_lo = x[:, :FH]
    h_hi = x[:, FH:]
    ep = e0.reshape(E2, 2 * DE)
    eye2 = jnp.eye(2, dtype=jnp.float32)
    for l in range(n_layers):
        last = l == n_layers - 1
        agg_lo, agg_hi = _agg_call(h_lo, h_hi, src, dst, an)
        h_lo, h_hi, hxl, hxh = _dense_calls[int(last)](
            agg_lo, agg_hi, h_lo, h_hi, W[l], Wg[l], Ug[l], Wxe[l])
        c0, c1 = _egather_call(hxl.reshape(-1), hxh.reshape(-1), src, dst)
        ep = _edge_call(ep, c0.reshape(E2, DE), c1.reshape(E2, DE),
                        jnp.kron(eye2, We[l]))
    h = jnp.concatenate([h_lo, h_hi], axis=1)
    return h, ep.reshape(E, DE)


def kernel(x, edge_index, A_norm, edge_attr, W, Wg, Ug, We, Wxe):
    src = edge_index[0].astype(jnp.int32)
    dst = edge_index[1].astype(jnp.int32)
    return _run(x, src, dst, A_norm, edge_attr, W, Wg, Ug, We, Wxe)
